# Initial kernel scaffold; baseline (speedup 1.0000x reference)
import jax
import jax.numpy as jnp
from jax.experimental import pallas as pl


def kernel(input_embeddings, centroids, biases, sW1, sb1, sW2, sb2, sW3, sb3, rW1, rb1, rW2, rb2, rW3, rb3):
    raise NotImplementedError("write your pallas kernel here")


# speedup vs baseline: 13.6301x; 13.6301x over previous
import functools

import jax
import jax.numpy as jnp
from jax import lax
from jax.experimental import pallas as pl
from jax.experimental.pallas import tpu as pltpu
from jax.experimental.pallas import tpu_sc as plsc

S = 2048
D = 1024
INNER = 256
E = 64
K = 8
CAP = 512
TOK = 256
NC = 2
NSUB = 16
NW = NC * NSUB
TPW = S // NW


def _router_body(x_ref, ct_ref, b_ref, w1_ref, b1_ref, w2_ref, b2_ref,
                 w3_ref, b3_ref, base_ref, aff_ref, gate_ref, topi_ref,
                 dsc_ref, dcb_ref, wexp_ref, cnt_ref):
    pid = pl.program_id(0)

    @pl.when(pid == 0)
    def _init():
        cnt_ref[...] = jnp.zeros_like(cnt_ref)

    x = x_ref[...]
    aff = jax.nn.sigmoid(
        jnp.dot(x, ct_ref[...], preferred_element_type=jnp.float32))
    aff_ref[...] = aff
    cur = aff + b_ref[...]
    lanes = lax.broadcasted_iota(jnp.int32, (TOK, E), 1)
    ams, avals, onehots = [], [], []
    for _ in range(K):
        mx = jnp.max(cur, axis=1, keepdims=True)
        am = jnp.min(jnp.where(cur == mx, lanes, E), axis=1, keepdims=True)
        oh = lanes == am
        avals.append(jnp.sum(jnp.where(oh, aff, 0.0), axis=1, keepdims=True))
        ams.append(am)
        onehots.append(oh)
        cur = jnp.where(oh, -jnp.inf, cur)
    topi = jnp.concatenate(ams, axis=1)
    sel = jnp.concatenate(avals, axis=1)
    mx = jnp.max(sel, axis=1, keepdims=True)
    ex = jnp.exp(sel - mx)
    gate = ex / jnp.sum(ex, axis=1, keepdims=True)
    topi_ref[...] = topi
    gate_ref[...] = gate

    usage = jnp.zeros((TOK, E), jnp.int32)
    for oh in onehots:
        usage = usage + oh.astype(jnp.int32)
    incl = usage
    shift = 1
    while shift < TOK:
        incl = incl + jnp.concatenate(
            [jnp.zeros((shift, E), jnp.int32), incl[:TOK - shift]], axis=0)
        shift *= 2
    basecnt = (incl - usage) + cnt_ref[0:1, :]
    dsc_cols, dcb_cols, wc_cols = [], [], []
    for k in range(K):
        slot = jnp.sum(jnp.where(onehots[k], basecnt, 0), axis=1,
                       keepdims=True)
        dest = ams[k] * CAP + slot
        valid = slot < CAP
        dsc_cols.append(jnp.where(valid, dest, E * CAP))
        dcb_cols.append(jnp.where(valid, dest, E * CAP))
        wc_cols.append(jnp.where(valid, gate[:, k:k + 1], 0.0))
    dsc_ref[...] = jnp.concatenate(dsc_cols, axis=1)
    dcb_ref[...] = jnp.concatenate(dcb_cols, axis=1)
    wexp_ref[...] = jnp.broadcast_to(
        jnp.concatenate(wc_cols, axis=1)[:, :, None], (TOK, K, 128))
    cnt_ref[0:1, :] = cnt_ref[0:1, :] + incl[TOK - 1:TOK, :]

    a = jnp.dot(x, w1_ref[...], preferred_element_type=jnp.float32) + b1_ref[...]
    u = jnp.dot(x, w3_ref[...], preferred_element_type=jnp.float32) + b3_ref[...]
    h = (a * jax.nn.sigmoid(a)) * u
    sh = jnp.dot(h, w2_ref[...], preferred_element_type=jnp.float32) + b2_ref[...]
    base_ref[...] = x + sh


def _run_router(x2d, ct, b2d, w1, b1, w2, b2, w3, b3):
    return pl.pallas_call(
        _router_body,
        grid=(S // TOK,),
        in_specs=[
            pl.BlockSpec((TOK, D), lambda i: (i, 0)),
            pl.BlockSpec((D, E), lambda i: (0, 0)),
            pl.BlockSpec((1, E), lambda i: (0, 0)),
            pl.BlockSpec((D, INNER), lambda i: (0, 0)),
            pl.BlockSpec((1, INNER), lambda i: (0, 0)),
            pl.BlockSpec((INNER, D), lambda i: (0, 0)),
            pl.BlockSpec((1, D), lambda i: (0, 0)),
            pl.BlockSpec((D, INNER), lambda i: (0, 0)),
            pl.BlockSpec((1, INNER), lambda i: (0, 0)),
        ],
        out_specs=[
            pl.BlockSpec((TOK, D), lambda i: (i, 0)),
            pl.BlockSpec((TOK, E), lambda i: (i, 0)),
            pl.BlockSpec((TOK, K), lambda i: (i, 0)),
            pl.BlockSpec((TOK, K), lambda i: (i, 0)),
            pl.BlockSpec((TOK, K), lambda i: (i, 0)),
            pl.BlockSpec((TOK, K), lambda i: (i, 0)),
            pl.BlockSpec((TOK, K, 128), lambda i: (i, 0, 0)),
        ],
        out_shape=[
            jax.ShapeDtypeStruct((S, D), jnp.float32),
            jax.ShapeDtypeStruct((S, E), jnp.float32),
            jax.ShapeDtypeStruct((S, K), jnp.float32),
            jax.ShapeDtypeStruct((S, K), jnp.int32),
            jax.ShapeDtypeStruct((S, K), jnp.int32),
            jax.ShapeDtypeStruct((S, K), jnp.int32),
            jax.ShapeDtypeStruct((S, K, 128), jnp.float32),
        ],
        scratch_shapes=[pltpu.VMEM((8, E), jnp.int32)],
    )(x2d, ct, b2d, w1, b1, w2, b2, w3, b3)


def _ffn_body(xd_ref, wd_ref, w1_ref, b1_ref, w2_ref, b2_ref, w3_ref, b3_ref,
              y_ref):
    pid = pl.program_id(0)

    @pl.when(pid < E)
    def _compute():
        x = xd_ref[...]
        a = jnp.dot(x, w1_ref[0],
                    preferred_element_type=jnp.float32) + b1_ref[0]
        u = jnp.dot(x, w3_ref[0],
                    preferred_element_type=jnp.float32) + b3_ref[0]
        h = (a * jax.nn.sigmoid(a)) * u
        y = jnp.dot(h, w2_ref[0],
                    preferred_element_type=jnp.float32) + b2_ref[0]
        y_ref[...] = y * wd_ref[...][:, 0:1]

    @pl.when(pid == E)
    def _zero_pad():
        y_ref[...] = jnp.zeros_like(y_ref)


def _run_ffn(disp, wdisp, rW1, rb1, rW2, rb2, rW3, rb3):
    def clamped(*unit):
        def index_map(e):
            return (jnp.minimum(e, E - 1),) + unit
        return index_map

    return pl.pallas_call(
        _ffn_body,
        grid=(E + 1,),
        in_specs=[
            pl.BlockSpec((CAP, D), clamped(0)),
            pl.BlockSpec((CAP, 128), clamped(0)),
            pl.BlockSpec((1, D, INNER), clamped(0, 0)),
            pl.BlockSpec((1, 1, INNER), clamped(0, 0)),
            pl.BlockSpec((1, INNER, D), clamped(0, 0)),
            pl.BlockSpec((1, 1, D), clamped(0, 0)),
            pl.BlockSpec((1, D, INNER), clamped(0, 0)),
            pl.BlockSpec((1, 1, INNER), clamped(0, 0)),
        ],
        out_specs=pl.BlockSpec((CAP, D), lambda e: (e, 0)),
        out_shape=jax.ShapeDtypeStruct((E * CAP + CAP, D), jnp.float32),
    )(disp, wdisp, rW1, rb1.reshape(E, 1, INNER), rW2, rb2.reshape(E, 1, D),
      rW3, rb3.reshape(E, 1, INNER))


def _run_dispatch(x2d, dsc_t, wexp_t):
    mesh = plsc.VectorSubcoreMesh(core_axis_name="c", subcore_axis_name="s")
    chunk = 32

    @functools.partial(
        pl.kernel,
        mesh=mesh,
        out_type=[jax.ShapeDtypeStruct((E * CAP + 8, D), jnp.float32),
                  jax.ShapeDtypeStruct((E * CAP + 8, 128), jnp.float32)],
        scratch_types=(
            [pltpu.VMEM((chunk, D), jnp.float32)]
            + [pltpu.VMEM((chunk,), jnp.int32) for _ in range(K)]
            + [pltpu.VMEM((chunk, 128), jnp.float32) for _ in range(K)]
            + [pltpu.SemaphoreType.DMA]
        ),
    )
    def body(x_hbm, dsc_hbm, wexp_hbm, disp_hbm, wdisp_hbm, x_v, *rest):
        idx_vs, w_vs, sem = rest[:K], rest[K:2 * K], rest[2 * K]
        wid = lax.axis_index("s") * NC + lax.axis_index("c")
        for c in range(TPW // chunk):
            t0 = wid * TPW + c * chunk
            pltpu.sync_copy(x_hbm.at[pl.ds(t0, chunk)], x_v)
            for k in range(K):
                pltpu.sync_copy(dsc_hbm.at[k, pl.ds(t0, chunk)], idx_vs[k])
                pltpu.sync_copy(wexp_hbm.at[k, pl.ds(t0, chunk)], w_vs[k])
            cps = [pltpu.async_copy(x_v, disp_hbm.at[idx_vs[k]], sem)
                   for k in range(K)]
            cps += [pltpu.async_copy(w_vs[k], wdisp_hbm.at[idx_vs[k]], sem)
                    for k in range(K)]
            for cp in cps:
                cp.wait()

    return body(x2d, dsc_t, wexp_t)


def _run_combine(base2d, ybuf, dcb_t):
    mesh = plsc.VectorSubcoreMesh(core_axis_name="c", subcore_axis_name="s")
    T = 8

    @functools.partial(
        pl.kernel,
        mesh=mesh,
        out_type=jax.ShapeDtypeStruct((S, D), jnp.float32),
        scratch_types=(
            [pltpu.VMEM((K * T, D), jnp.float32),
             pltpu.VMEM((T, D), jnp.float32),
             pltpu.VMEM((T, D), jnp.float32)]
            + [pltpu.VMEM((T,), jnp.int32) for _ in range(K)]
            + [pltpu.SemaphoreType.DMA]
        ),
    )
    def body(base_hbm, y_hbm, dcb_hbm, hid_hbm, g_v, b_v, o_v, *rest):
        idx_vs, sem = rest[:K], rest[K]
        wid = lax.axis_index("s") * NC + lax.axis_index("c")

        def chunk_body(c, carry):
            t0 = wid * TPW + c * T
            pltpu.sync_copy(base_hbm.at[pl.ds(t0, T)], b_v)
            for k in range(K):
                pltpu.sync_copy(dcb_hbm.at[k, pl.ds(t0, T)], idx_vs[k])
            cps = [pltpu.async_copy(y_hbm.at[idx_vs[k]],
                                    g_v.at[pl.ds(k * T, T)], sem)
                   for k in range(K)]
            for cp in cps:
                cp.wait()
            for t in range(T):

                def vec_body(v, inner, _t=t):
                    sl = pl.ds(v * 16, 16)
                    acc = b_v[_t, sl]
                    for k in range(K):
                        acc = acc + g_v[k * T + _t, sl]
                    o_v[_t, sl] = acc
                    return inner

                lax.fori_loop(0, D // 16, vec_body, 0)
            pltpu.sync_copy(o_v, hid_hbm.at[pl.ds(t0, T)])
            return carry

        lax.fori_loop(0, TPW // T, chunk_body, 0)

    return body(base2d, ybuf, dcb_t)


def kernel(input_embeddings, centroids, biases, sW1, sb1, sW2, sb2, sW3, sb3,
           rW1, rb1, rW2, rb2, rW3, rb3):
    x2d = input_embeddings.reshape(S, D)
    base, aff, gate, topi, dsc, dcb, wexp = _run_router(
        x2d, centroids.T, biases.reshape(1, E),
        sW1[0], sb1, sW2[0], sb2, sW3[0], sb3)
    disp, wdisp = _run_dispatch(x2d, dsc.T, wexp.transpose(1, 0, 2))
    ybuf = _run_ffn(disp, wdisp, rW1, rb1, rW2, rb2, rW3, rb3)
    hid = _run_combine(base, ybuf, dcb.T)
    return (hid.reshape(1, S, D), aff.reshape(1, S, E),
            gate.reshape(1, S, K), topi.reshape(1, S, K))


# speedup vs baseline: 13.7026x; 1.0053x over previous
import functools

import jax
import jax.numpy as jnp
from jax import lax
from jax.experimental import pallas as pl
from jax.experimental.pallas import tpu as pltpu
from jax.experimental.pallas import tpu_sc as plsc

S = 2048
D = 1024
INNER = 256
E = 64
K = 8
CAP = 512
TOK = 256
NC = 2
NSUB = 16
NW = NC * NSUB
TPW = S // NW


def _router_body(x_ref, ct_ref, b_ref, w1_ref, b1_ref, w2_ref, b2_ref,
                 w3_ref, b3_ref, base_ref, aff_ref, gate_ref, topi_ref,
                 dsc_ref, dcb_ref, wexp_ref, cnt_ref):
    pid = pl.program_id(0)

    @pl.when(pid == 0)
    def _init():
        cnt_ref[...] = jnp.zeros_like(cnt_ref)

    x = x_ref[...]
    aff = jax.nn.sigmoid(
        jnp.dot(x, ct_ref[...], preferred_element_type=jnp.float32))
    aff_ref[...] = aff
    cur = aff + b_ref[...]
    lanes = lax.broadcasted_iota(jnp.int32, (TOK, E), 1)
    ams, avals, onehots = [], [], []
    for _ in range(K):
        mx = jnp.max(cur, axis=1, keepdims=True)
        am = jnp.min(jnp.where(cur == mx, lanes, E), axis=1, keepdims=True)
        oh = lanes == am
        avals.append(jnp.sum(jnp.where(oh, aff, 0.0), axis=1, keepdims=True))
        ams.append(am)
        onehots.append(oh)
        cur = jnp.where(oh, -jnp.inf, cur)
    topi = jnp.concatenate(ams, axis=1)
    sel = jnp.concatenate(avals, axis=1)
    mx = jnp.max(sel, axis=1, keepdims=True)
    ex = jnp.exp(sel - mx)
    gate = ex / jnp.sum(ex, axis=1, keepdims=True)
    topi_ref[...] = topi
    gate_ref[...] = gate

    usage = jnp.zeros((TOK, E), jnp.int32)
    for oh in onehots:
        usage = usage + oh.astype(jnp.int32)
    incl = usage
    shift = 1
    while shift < TOK:
        incl = incl + jnp.concatenate(
            [jnp.zeros((shift, E), jnp.int32), incl[:TOK - shift]], axis=0)
        shift *= 2
    basecnt = (incl - usage) + cnt_ref[0:1, :]
    dsc_cols, dcb_cols, wc_cols = [], [], []
    for k in range(K):
        slot = jnp.sum(jnp.where(onehots[k], basecnt, 0), axis=1,
                       keepdims=True)
        dest = ams[k] * CAP + slot
        valid = slot < CAP
        dsc_cols.append(jnp.where(valid, dest, E * CAP))
        dcb_cols.append(jnp.where(valid, dest, E * CAP))
        wc_cols.append(jnp.where(valid, gate[:, k:k + 1], 0.0))
    dsc_ref[...] = jnp.concatenate(dsc_cols, axis=1)
    dcb_ref[...] = jnp.concatenate(dcb_cols, axis=1)
    wexp_ref[...] = jnp.broadcast_to(
        jnp.concatenate(wc_cols, axis=1)[:, :, None], (TOK, K, 128))
    cnt_ref[0:1, :] = cnt_ref[0:1, :] + incl[TOK - 1:TOK, :]

    xb = x.astype(jnp.bfloat16)
    a = jnp.dot(xb, w1_ref[...].astype(jnp.bfloat16),
                preferred_element_type=jnp.float32) + b1_ref[...]
    u = jnp.dot(xb, w3_ref[...].astype(jnp.bfloat16),
                preferred_element_type=jnp.float32) + b3_ref[...]
    h = ((a * jax.nn.sigmoid(a)) * u).astype(jnp.bfloat16)
    sh = jnp.dot(h, w2_ref[...].astype(jnp.bfloat16),
                 preferred_element_type=jnp.float32) + b2_ref[...]
    base_ref[...] = x + sh


def _run_router(x2d, ct, b2d, w1, b1, w2, b2, w3, b3):
    return pl.pallas_call(
        _router_body,
        grid=(S // TOK,),
        in_specs=[
            pl.BlockSpec((TOK, D), lambda i: (i, 0)),
            pl.BlockSpec((D, E), lambda i: (0, 0)),
            pl.BlockSpec((1, E), lambda i: (0, 0)),
            pl.BlockSpec((D, INNER), lambda i: (0, 0)),
            pl.BlockSpec((1, INNER), lambda i: (0, 0)),
            pl.BlockSpec((INNER, D), lambda i: (0, 0)),
            pl.BlockSpec((1, D), lambda i: (0, 0)),
            pl.BlockSpec((D, INNER), lambda i: (0, 0)),
            pl.BlockSpec((1, INNER), lambda i: (0, 0)),
        ],
        out_specs=[
            pl.BlockSpec((TOK, D), lambda i: (i, 0)),
            pl.BlockSpec((TOK, E), lambda i: (i, 0)),
            pl.BlockSpec((TOK, K), lambda i: (i, 0)),
            pl.BlockSpec((TOK, K), lambda i: (i, 0)),
            pl.BlockSpec((TOK, K), lambda i: (i, 0)),
            pl.BlockSpec((TOK, K), lambda i: (i, 0)),
            pl.BlockSpec((TOK, K, 128), lambda i: (i, 0, 0)),
        ],
        out_shape=[
            jax.ShapeDtypeStruct((S, D), jnp.float32),
            jax.ShapeDtypeStruct((S, E), jnp.float32),
            jax.ShapeDtypeStruct((S, K), jnp.float32),
            jax.ShapeDtypeStruct((S, K), jnp.int32),
            jax.ShapeDtypeStruct((S, K), jnp.int32),
            jax.ShapeDtypeStruct((S, K), jnp.int32),
            jax.ShapeDtypeStruct((S, K, 128), jnp.float32),
        ],
        scratch_shapes=[pltpu.VMEM((8, E), jnp.int32)],
    )(x2d, ct, b2d, w1, b1, w2, b2, w3, b3)


def _ffn_body(xd_ref, wd_ref, w1_ref, b1_ref, w2_ref, b2_ref, w3_ref, b3_ref,
              y_ref):
    pid = pl.program_id(0)

    @pl.when(pid < E)
    def _compute():
        x = xd_ref[...].astype(jnp.bfloat16)
        a = jnp.dot(x, w1_ref[0].astype(jnp.bfloat16),
                    preferred_element_type=jnp.float32) + b1_ref[0]
        u = jnp.dot(x, w3_ref[0].astype(jnp.bfloat16),
                    preferred_element_type=jnp.float32) + b3_ref[0]
        h = ((a * jax.nn.sigmoid(a)) * u).astype(jnp.bfloat16)
        y = jnp.dot(h, w2_ref[0].astype(jnp.bfloat16),
                    preferred_element_type=jnp.float32) + b2_ref[0]
        y_ref[...] = y * wd_ref[...][:, 0:1]

    @pl.when(pid == E)
    def _zero_pad():
        y_ref[...] = jnp.zeros_like(y_ref)


def _run_ffn(disp, wdisp, rW1, rb1, rW2, rb2, rW3, rb3):
    def clamped(*unit):
        def index_map(e):
            return (jnp.minimum(e, E - 1),) + unit
        return index_map

    return pl.pallas_call(
        _ffn_body,
        grid=(E + 1,),
        in_specs=[
            pl.BlockSpec((CAP, D), clamped(0)),
            pl.BlockSpec((CAP, 128), clamped(0)),
            pl.BlockSpec((1, D, INNER), clamped(0, 0)),
            pl.BlockSpec((1, 1, INNER), clamped(0, 0)),
            pl.BlockSpec((1, INNER, D), clamped(0, 0)),
            pl.BlockSpec((1, 1, D), clamped(0, 0)),
            pl.BlockSpec((1, D, INNER), clamped(0, 0)),
            pl.BlockSpec((1, 1, INNER), clamped(0, 0)),
        ],
        out_specs=pl.BlockSpec((CAP, D), lambda e: (e, 0)),
        out_shape=jax.ShapeDtypeStruct((E * CAP + CAP, D), jnp.float32),
    )(disp, wdisp, rW1, rb1.reshape(E, 1, INNER), rW2, rb2.reshape(E, 1, D),
      rW3, rb3.reshape(E, 1, INNER))


def _run_dispatch(x2d, dsc_t, wexp_t):
    mesh = plsc.VectorSubcoreMesh(core_axis_name="c", subcore_axis_name="s")
    chunk = 32

    @functools.partial(
        pl.kernel,
        mesh=mesh,
        out_type=[jax.ShapeDtypeStruct((E * CAP + 8, D), jnp.float32),
                  jax.ShapeDtypeStruct((E * CAP + 8, 128), jnp.float32)],
        scratch_types=(
            [pltpu.VMEM((chunk, D), jnp.float32)]
            + [pltpu.VMEM((chunk,), jnp.int32) for _ in range(K)]
            + [pltpu.VMEM((chunk, 128), jnp.float32) for _ in range(K)]
            + [pltpu.SemaphoreType.DMA]
        ),
    )
    def body(x_hbm, dsc_hbm, wexp_hbm, disp_hbm, wdisp_hbm, x_v, *rest):
        idx_vs, w_vs, sem = rest[:K], rest[K:2 * K], rest[2 * K]
        wid = lax.axis_index("s") * NC + lax.axis_index("c")
        for c in range(TPW // chunk):
            t0 = wid * TPW + c * chunk
            pltpu.sync_copy(x_hbm.at[pl.ds(t0, chunk)], x_v)
            for k in range(K):
                pltpu.sync_copy(dsc_hbm.at[k, pl.ds(t0, chunk)], idx_vs[k])
                pltpu.sync_copy(wexp_hbm.at[k, pl.ds(t0, chunk)], w_vs[k])
            cps = [pltpu.async_copy(x_v, disp_hbm.at[idx_vs[k]], sem)
                   for k in range(K)]
            cps += [pltpu.async_copy(w_vs[k], wdisp_hbm.at[idx_vs[k]], sem)
                    for k in range(K)]
            for cp in cps:
                cp.wait()

    return body(x2d, dsc_t, wexp_t)


def _run_combine(base2d, ybuf, dcb_t):
    mesh = plsc.VectorSubcoreMesh(core_axis_name="c", subcore_axis_name="s")
    T = 8

    @functools.partial(
        pl.kernel,
        mesh=mesh,
        out_type=jax.ShapeDtypeStruct((S, D), jnp.float32),
        scratch_types=(
            [pltpu.VMEM((K * T, D), jnp.float32),
             pltpu.VMEM((T, D), jnp.float32),
             pltpu.VMEM((T, D), jnp.float32)]
            + [pltpu.VMEM((T,), jnp.int32) for _ in range(K)]
            + [pltpu.SemaphoreType.DMA]
        ),
    )
    def body(base_hbm, y_hbm, dcb_hbm, hid_hbm, g_v, b_v, o_v, *rest):
        idx_vs, sem = rest[:K], rest[K]
        wid = lax.axis_index("s") * NC + lax.axis_index("c")

        def chunk_body(c, carry):
            t0 = wid * TPW + c * T
            pltpu.sync_copy(base_hbm.at[pl.ds(t0, T)], b_v)
            for k in range(K):
                pltpu.sync_copy(dcb_hbm.at[k, pl.ds(t0, T)], idx_vs[k])
            cps = [pltpu.async_copy(y_hbm.at[idx_vs[k]],
                                    g_v.at[pl.ds(k * T, T)], sem)
                   for k in range(K)]
            for cp in cps:
                cp.wait()
            for t in range(T):

                def vec_body(v, inner, _t=t):
                    sl = pl.ds(v * 16, 16)
                    acc = b_v[_t, sl]
                    for k in range(K):
                        acc = acc + g_v[k * T + _t, sl]
                    o_v[_t, sl] = acc
                    return inner

                lax.fori_loop(0, D // 16, vec_body, 0)
            pltpu.sync_copy(o_v, hid_hbm.at[pl.ds(t0, T)])
            return carry

        lax.fori_loop(0, TPW // T, chunk_body, 0)

    return body(base2d, ybuf, dcb_t)


def kernel(input_embeddings, centroids, biases, sW1, sb1, sW2, sb2, sW3, sb3,
           rW1, rb1, rW2, rb2, rW3, rb3):
    x2d = input_embeddings.reshape(S, D)
    base, aff, gate, topi, dsc, dcb, wexp = _run_router(
        x2d, centroids.T, biases.reshape(1, E),
        sW1[0], sb1, sW2[0], sb2, sW3[0], sb3)
    disp, wdisp = _run_dispatch(x2d, dsc.T, wexp.transpose(1, 0, 2))
    ybuf = _run_ffn(disp, wdisp, rW1, rb1, rW2, rb2, rW3, rb3)
    hid = _run_combine(base, ybuf, dcb.T)
    return (hid.reshape(1, S, D), aff.reshape(1, S, E),
            gate.reshape(1, S, K), topi.reshape(1, S, K))
